# baseline (speedup 1.0000x reference)
import functools

import jax
import jax.numpy as jnp
from jax import lax
from jax.experimental import pallas as pl
from jax.experimental.pallas import tpu as pltpu
from jax.experimental.pallas import tpu_sc as plsc

_B = 16384
_D = 64
_NC = 2
_NS = 16
_NW = _NC * _NS
_BPW = _B // _NW
_C = 1
_NSET = 8
_NGRP = _BPW // 16
_L = 16


@functools.cache
def _gather_kernel():
    mesh = plsc.VectorSubcoreMesh(
        core_axis_name="c", subcore_axis_name="s",
        num_cores=_NC, num_subcores=_NS,
    )

    block_types = [
        pltpu.VMEM((_D, 128), jnp.float32) for _ in range(_NSET * _C)
    ]

    @functools.partial(
        pl.kernel,
        out_type=jax.ShapeDtypeStruct((_D, _B), jnp.float32),
        mesh=mesh,
        scratch_types=[
            pltpu.VMEM((_BPW,), jnp.int32),
            *block_types,
            pltpu.VMEM((_D, _BPW), jnp.float32),
            pltpu.SemaphoreType.DMA,
            *[pltpu.SemaphoreType.DMA for _ in range(_NSET)],
        ],
        compiler_params=pltpu.CompilerParams(needs_layout_passes=False),
    )
    def body(idx_hbm, tableT_hbm, outT_hbm, lab_v, *rest):
        bufs = [
            rest[s * _C:(s + 1) * _C] for s in range(_NSET)
        ]
        outT_v = rest[_NSET * _C]
        sem_in = rest[_NSET * _C + 1]
        sems = rest[_NSET * _C + 2:]
        wid = lax.axis_index("s") * _NC + lax.axis_index("c")
        base = wid * _BPW
        pltpu.async_copy(idx_hbm.at[pl.ds(base, _BPW)], lab_v, sem_in).wait()

        lane = lax.iota(jnp.int32, _L)
        zeros = jnp.zeros((_L,), jnp.int32)
        nchunk = _L // _C

        def fire(lab16, lbase, s):
            for e in range(_C):
                blk0 = pl.multiple_of(
                    lax.bitwise_and(lab16[lbase + e], -128), 128
                )
                pltpu.async_copy(
                    tableT_hbm.at[:, pl.ds(blk0, 128)], bufs[s][e], sems[s]
                )

        def drain(s):
            for e in range(_C):
                pltpu.make_async_copy(
                    tableT_hbm.at[:, pl.ds(0, 128)], bufs[s][e], sems[s]
                ).wait()

        def extract(lab16, lbase, off, s):
            for e in range(_C):
                l_vec = zeros + lax.bitwise_and(lab16[lbase + e], 127)
                p_vec = zeros + (off + e)
                for c in range(_D // _L):
                    val = plsc.load_gather(bufs[s][e], [c * _L + lane, l_vec])
                    plsc.store_scatter(outT_v, [c * _L + lane, p_vec], val)

        lab0 = lab_v[pl.ds(0, _L)]
        for j in range(_NSET - 1):
            fire(lab0, j * _C, j)

        def do_group(g, _):
            lab16 = lab_v[pl.ds(g * _L, _L)]
            off = g * _L
            for j in range(nchunk):
                s = j % _NSET
                fj = j + _NSET - 1
                if fj < nchunk:
                    fire(lab16, fj * _C, (fj % _NSET))
                else:
                    fj -= nchunk

                    @pl.when(g < _NGRP - 1)
                    def _(fj=fj):
                        lab_n = lab_v[pl.ds((g + 1) * _L, _L)]
                        fire(lab_n, fj * _C, fj % _NSET)

                drain(s)
                extract(lab16, j * _C, off + j * _C, s)
            return ()

        lax.fori_loop(0, _NGRP, do_group, ())
        pltpu.sync_copy(outT_v, outT_hbm.at[:, pl.ds(base, _BPW)])

    return body


def kernel(class_label, table, uncond_table):
    del uncond_table
    idx = class_label.astype(jnp.int32)
    outT = _gather_kernel()(idx, table.T)
    return outT.T.reshape(_B, 1, _D)
